# baseline (speedup 1.0000x reference)
import functools

import jax
import jax.numpy as jnp
from jax import lax
from jax.experimental import pallas as pl
from jax.experimental.pallas import tpu as pltpu


def _conv_stats_kernel(x_ref, w_ref, y_ref, s_ref, *,
                       kh, kw, ho, wo, wp, cin, cout, rows_c):
    tt = pl.program_id(1)
    nc = ho // rows_c
    cw = rows_c * wp
    s0 = jnp.zeros((cout, 1), jnp.float32)
    s1 = jnp.zeros((cout, 1), jnp.float32)
    lane = lax.broadcasted_iota(jnp.int32, (cout, cw), 1)
    valid = (lane % wp) < wo
    for c in range(nc):
        acc = jnp.zeros((kw * cout, cw), jnp.float32)
        for dt in range(3):
            for dh in range(kh):
                r0 = c * rows_c + dh
                p = x_ref[tt + dt, r0:r0 + rows_c].reshape(cw, cin)
                acc = acc + lax.dot_general(
                    w_ref[dt * kh + dh], p,
                    (((0,), (1,)), ((), ())),
                    preferred_element_type=jnp.float32)
        m = acc[0:cout]
        for dw in range(1, kw):
            m = m + jnp.roll(acc[dw * cout:(dw + 1) * cout], -dw, axis=1)
        y_ref[:, c * cw:(c + 1) * cw] = m.astype(jnp.bfloat16)
        mm = jnp.where(valid, m, 0.0)
        s0 = s0 + jnp.sum(mm, axis=1, keepdims=True)
        s1 = s1 + jnp.sum(mm * mm, axis=1, keepdims=True)
    s_ref[:, 0:1] = s0
    s_ref[:, 1:2] = s1


def _bn_relu_kernel(y_ref, sc_ref, sh_ref, o_ref, *, wo):
    v = y_ref[...].astype(jnp.float32)
    o_ref[...] = jnp.maximum(
        v[:, :, :wo] * sc_ref[...] + sh_ref[...], 0.0)


def kernel(x_pt, w_pt, gamma, beta):
    n, cin, t, h, w = x_pt.shape
    cout = w_pt.shape[0]
    kt, kh, kw = w_pt.shape[2:]
    eps = 1e-5

    wp = 16
    while wp < w + 2 or (h * wp) % 128 != 0:
        wp += 16
    hww = h * wp

    xw = jnp.pad(jnp.transpose(x_pt, (0, 2, 3, 4, 1)).astype(jnp.float32),
                 [(0, 0), (1, 1), (1, 1), (1, wp - w - 1), (0, 0)])
    xw = xw.astype(jnp.bfloat16)
    wt = jnp.transpose(w_pt, (2, 3, 1, 4, 0)).reshape(kt * kh, cin, kw * cout)
    wt = wt.astype(jnp.bfloat16)
    hp, tp = h + 2, t + 2

    rows_c = h
    for cand in range(1, h + 1):
        if (h % cand == 0 and (cand * wp) % 128 == 0
                and cand * wp * kw * cout * 4 <= 760_000):
            rows_c = cand
            break

    y, stats = pl.pallas_call(
        functools.partial(_conv_stats_kernel, kh=kh, kw=kw, ho=h, wo=w,
                          wp=wp, cin=cin, cout=cout, rows_c=rows_c),
        grid=(n, t),
        in_specs=[
            pl.BlockSpec((None, tp, hp, wp, cin),
                         lambda b, tt: (b, 0, 0, 0, 0)),
            pl.BlockSpec((kt * kh, cin, kw * cout),
                         lambda b, tt: (0, 0, 0)),
        ],
        out_specs=(
            pl.BlockSpec((None, cout, hww), lambda b, tt: (b, 0, tt)),
            pl.BlockSpec((None, None, cout, 2), lambda b, tt: (b, tt, 0, 0)),
        ),
        out_shape=(
            jax.ShapeDtypeStruct((n, cout, t * hww), jnp.bfloat16),
            jax.ShapeDtypeStruct((n, t, cout, 2), jnp.float32),
        ),
        compiler_params=pltpu.CompilerParams(
            dimension_semantics=("parallel", "parallel"),
            vmem_limit_bytes=56 * 1024 * 1024,
        ),
    )(xw, wt)

    s = jnp.sum(stats, axis=(0, 1))
    count = float(n * t * h * w)
    mean = s[:, 0] / count
    var = jnp.maximum(s[:, 1] / count - mean * mean, 0.0)
    scale = gamma.astype(jnp.float32) * lax.rsqrt(var + eps)
    shift = beta.astype(jnp.float32) - mean * scale
    sc = jnp.broadcast_to(scale.reshape(cout, 1, 1), (cout, h, w))
    sh = jnp.broadcast_to(shift.reshape(cout, 1, 1), (cout, h, w))

    y5 = y.reshape(n, cout, t, h, wp)

    out = pl.pallas_call(
        functools.partial(_bn_relu_kernel, wo=w),
        grid=(n, t),
        in_specs=[
            pl.BlockSpec((None, cout, None, h, wp),
                         lambda b, tt: (b, 0, tt, 0, 0)),
            pl.BlockSpec((cout, h, w), lambda b, tt: (0, 0, 0)),
            pl.BlockSpec((cout, h, w), lambda b, tt: (0, 0, 0)),
        ],
        out_specs=pl.BlockSpec((None, cout, None, h, w),
                               lambda b, tt: (b, 0, tt, 0, 0)),
        out_shape=jax.ShapeDtypeStruct((n, cout, t, h, w), jnp.float32),
        compiler_params=pltpu.CompilerParams(
            dimension_semantics=("parallel", "parallel"),
            vmem_limit_bytes=48 * 1024 * 1024,
        ),
    )(y5, sc, sh)

    return out
